# Initial kernel scaffold; baseline (speedup 1.0000x reference)
import jax
import jax.numpy as jnp
from jax.experimental import pallas as pl


def kernel(voro_x, edge_index, batch, W1, Wconv, W_ih, W_hh, b_ih, b_hh, Wp, bp):
    raise NotImplementedError("write your pallas kernel here")


# speedup vs baseline: 6.7334x; 6.7334x over previous
import functools

import jax
import jax.numpy as jnp
from jax import lax
from jax.experimental import pallas as pl
from jax.experimental.pallas import tpu as pltpu
from jax.experimental.pallas import tpu_sc as plsc

N = 10000
E = 320000
F = 128
H = 128
STEPS = 4
G = 64

ROWS_BLK = 1000
GRID = N // ROWS_BLK

NW = 32
NCH = 100
CH = 100
NP = 10240
TILE_ROWS = NP // 16


def _gru(p0, p1, h, gh, wih, bih):
    agg = p0 + p1
    gi = jnp.dot(agg, wih, preferred_element_type=jnp.float32) + bih
    r = jax.nn.sigmoid(gi[:, 0:H] + gh[:, 0:H])
    z = jax.nn.sigmoid(gi[:, H:2 * H] + gh[:, H:2 * H])
    n = jnp.tanh(gi[:, 2 * H:3 * H] + r * gh[:, 2 * H:3 * H])
    return (1.0 - z) * n + z * h


def _tc_init(voro_x, W1t, Wc0, W_hht, b_hh2):
    def body(x_ref, w1_ref, wc_ref, whh_ref, bhh_ref, h_ref, m_ref, gh_ref):
        h = jnp.tanh(jnp.dot(x_ref[...], w1_ref[...],
                             preferred_element_type=jnp.float32))
        h_ref[...] = h
        m_ref[...] = jnp.dot(h, wc_ref[...], preferred_element_type=jnp.float32)
        gh_ref[...] = jnp.dot(h, whh_ref[...],
                              preferred_element_type=jnp.float32) + bhh_ref[...]

    return pl.pallas_call(
        body,
        grid=(GRID,),
        in_specs=[
            pl.BlockSpec((ROWS_BLK, F), lambda i: (i, 0)),
            pl.BlockSpec((F, H), lambda i: (0, 0)),
            pl.BlockSpec((H, H), lambda i: (0, 0)),
            pl.BlockSpec((H, 3 * H), lambda i: (0, 0)),
            pl.BlockSpec((1, 3 * H), lambda i: (0, 0)),
        ],
        out_specs=[
            pl.BlockSpec((ROWS_BLK, H), lambda i: (i, 0)),
            pl.BlockSpec((ROWS_BLK, H), lambda i: (i, 0)),
            pl.BlockSpec((ROWS_BLK, 3 * H), lambda i: (i, 0)),
        ],
        out_shape=[
            jax.ShapeDtypeStruct((N, H), jnp.float32),
            jax.ShapeDtypeStruct((N, H), jnp.float32),
            jax.ShapeDtypeStruct((N, 3 * H), jnp.float32),
        ],
    )(voro_x, W1t, Wc0, W_hht, b_hh2)


def _tc_step(p0, p1, h, gh, W_iht, b_ih2, Wc_next, W_hht, b_hh2):
    def body(p0_ref, p1_ref, h_ref, gh_ref, wih_ref, bih_ref, wc_ref,
             whh_ref, bhh_ref, hn_ref, m_ref, ghn_ref):
        h_new = _gru(p0_ref[...], p1_ref[...], h_ref[...], gh_ref[...],
                     wih_ref[...], bih_ref[...])
        hn_ref[...] = h_new
        m_ref[...] = jnp.dot(h_new, wc_ref[...],
                             preferred_element_type=jnp.float32)
        ghn_ref[...] = jnp.dot(h_new, whh_ref[...],
                               preferred_element_type=jnp.float32) + bhh_ref[...]

    return pl.pallas_call(
        body,
        grid=(GRID,),
        in_specs=[
            pl.BlockSpec((ROWS_BLK, H), lambda i: (i, 0)),
            pl.BlockSpec((ROWS_BLK, H), lambda i: (i, 0)),
            pl.BlockSpec((ROWS_BLK, H), lambda i: (i, 0)),
            pl.BlockSpec((ROWS_BLK, 3 * H), lambda i: (i, 0)),
            pl.BlockSpec((H, 3 * H), lambda i: (0, 0)),
            pl.BlockSpec((1, 3 * H), lambda i: (0, 0)),
            pl.BlockSpec((H, H), lambda i: (0, 0)),
            pl.BlockSpec((H, 3 * H), lambda i: (0, 0)),
            pl.BlockSpec((1, 3 * H), lambda i: (0, 0)),
        ],
        out_specs=[
            pl.BlockSpec((ROWS_BLK, H), lambda i: (i, 0)),
            pl.BlockSpec((ROWS_BLK, H), lambda i: (i, 0)),
            pl.BlockSpec((ROWS_BLK, 3 * H), lambda i: (i, 0)),
        ],
        out_shape=[
            jax.ShapeDtypeStruct((N, H), jnp.float32),
            jax.ShapeDtypeStruct((N, H), jnp.float32),
            jax.ShapeDtypeStruct((N, 3 * H), jnp.float32),
        ],
    )(p0, p1, h, gh, W_iht, b_ih2, Wc_next, W_hht, b_hh2)


def _tc_final(p0, p1, h, gh, W_iht, b_ih2, batch_r, Wp, bp2):
    def body(p0_ref, p1_ref, h_ref, gh_ref, wih_ref, bih_ref, b_ref,
             wp_ref, bp_ref, out_ref, sums_ref, cnt_ref):
        i = pl.program_id(0)
        h_new = _gru(p0_ref[...], p1_ref[...], h_ref[...], gh_ref[...],
                     wih_ref[...], bih_ref[...])
        seg = b_ref[0]
        oh = (lax.broadcasted_iota(jnp.int32, (G, ROWS_BLK), 0)
              == seg).astype(jnp.float32)

        @pl.when(i == 0)
        def _():
            sums_ref[...] = jnp.zeros_like(sums_ref)
            cnt_ref[...] = jnp.zeros_like(cnt_ref)

        sums_ref[...] += jnp.dot(oh, h_new, preferred_element_type=jnp.float32)
        cnt_ref[...] += jnp.sum(oh, axis=1, keepdims=True)

        @pl.when(i == GRID - 1)
        def _():
            pooled = sums_ref[...] / jnp.maximum(cnt_ref[...], 1.0)
            act = jnp.maximum(pooled, 0.0)
            pred = lax.dot_general(
                wp_ref[...], act, (((1,), (1,)), ((), ())),
                preferred_element_type=jnp.float32) + bp_ref[...]
            out_ref[...] = pred

    return pl.pallas_call(
        body,
        grid=(GRID,),
        in_specs=[
            pl.BlockSpec((ROWS_BLK, H), lambda i: (i, 0)),
            pl.BlockSpec((ROWS_BLK, H), lambda i: (i, 0)),
            pl.BlockSpec((ROWS_BLK, H), lambda i: (i, 0)),
            pl.BlockSpec((ROWS_BLK, 3 * H), lambda i: (i, 0)),
            pl.BlockSpec((H, 3 * H), lambda i: (0, 0)),
            pl.BlockSpec((1, 3 * H), lambda i: (0, 0)),
            pl.BlockSpec((1, 1, ROWS_BLK), lambda i: (i, 0, 0)),
            pl.BlockSpec((1, H), lambda i: (0, 0)),
            pl.BlockSpec((1, 1), lambda i: (0, 0)),
        ],
        out_specs=pl.BlockSpec((1, G), lambda i: (0, 0)),
        out_shape=jax.ShapeDtypeStruct((1, G), jnp.float32),
        scratch_shapes=[
            pltpu.VMEM((G, H), jnp.float32),
            pltpu.VMEM((G, 1), jnp.float32),
        ],
    )(p0, p1, h, gh, W_iht, b_ih2, batch_r, Wp, bp2)


def _sc_agg(m, srcr, dstr, zeros):
    mesh = plsc.VectorSubcoreMesh(core_axis_name="c", subcore_axis_name="s")

    @functools.partial(
        pl.kernel,
        out_type=[
            jax.ShapeDtypeStruct((NP, H), jnp.float32),
            jax.ShapeDtypeStruct((NP, H), jnp.float32),
        ],
        mesh=mesh,
        scratch_types=[
            pltpu.VMEM((NCH, CH), jnp.int32),
            pltpu.VMEM((NCH, CH), jnp.int32),
            pltpu.VMEM((CH, H), jnp.float32),
            pltpu.VMEM_SHARED((NP, H), jnp.float32),
            pltpu.SemaphoreType.DMA,
        ],
    )
    def agg_k(m_hbm, src_hbm, dst_hbm, z_hbm, out0_hbm, out1_hbm,
              src_v, dst_v, rows_v, acc_sh, sem):
        cid = lax.axis_index("c")
        sid = lax.axis_index("s")
        wid = sid * 2 + cid
        pltpu.sync_copy(z_hbm, acc_sh.at[pl.ds(sid * TILE_ROWS, TILE_ROWS)])
        pltpu.sync_copy(src_hbm.at[wid], src_v)
        pltpu.sync_copy(dst_hbm.at[wid], dst_v)
        plsc.subcore_barrier()

        def body(j, c):
            pltpu.async_copy(m_hbm.at[src_v.at[j]], rows_v, sem).wait()
            pltpu.sync_copy(rows_v, acc_sh.at[dst_v.at[j]], add=True)
            return c

        lax.fori_loop(0, NCH, body, 0)
        plsc.subcore_barrier()

        @pl.when(cid == 0)
        def _():
            pltpu.sync_copy(acc_sh.at[pl.ds(sid * TILE_ROWS, TILE_ROWS)],
                            out0_hbm.at[pl.ds(sid * TILE_ROWS, TILE_ROWS)])

        @pl.when(cid == 1)
        def _():
            pltpu.sync_copy(acc_sh.at[pl.ds(sid * TILE_ROWS, TILE_ROWS)],
                            out1_hbm.at[pl.ds(sid * TILE_ROWS, TILE_ROWS)])

    return agg_k(m, srcr, dstr, zeros)


def kernel(voro_x, edge_index, batch, W1, Wconv, W_ih, W_hh, b_ih, b_hh, Wp, bp):
    W1t = W1.T
    W_iht = W_ih.T
    W_hht = W_hh.T
    b_ih2 = b_ih.reshape(1, 3 * H)
    b_hh2 = b_hh.reshape(1, 3 * H)
    src = edge_index[0].astype(jnp.int32).reshape(NW, NCH, CH)
    dst = edge_index[1].astype(jnp.int32).reshape(NW, NCH, CH)
    zeros = jnp.zeros((TILE_ROWS, H), jnp.float32)
    batch_r = batch.astype(jnp.int32).reshape(GRID, 1, ROWS_BLK)

    h, m, gh = _tc_init(voro_x, W1t, Wconv[0], W_hht, b_hh2)
    out2 = None
    for i in range(STEPS):
        p0, p1 = _sc_agg(m, src, dst, zeros)
        if i < STEPS - 1:
            h, m, gh = _tc_step(p0, p1, h, gh, W_iht, b_ih2, Wconv[i + 1],
                                W_hht, b_hh2)
        else:
            out2 = _tc_final(p0, p1, h, gh, W_iht, b_ih2, batch_r, Wp,
                             bp.reshape(1, 1))
    return out2.reshape(G)


# speedup vs baseline: 8.4606x; 1.2565x over previous
import functools

import jax
import jax.numpy as jnp
from jax import lax
from jax.experimental import pallas as pl
from jax.experimental.pallas import tpu as pltpu
from jax.experimental.pallas import tpu_sc as plsc

N = 10000
E = 320000
F = 128
H = 128
STEPS = 4
G = 64

ROWS_BLK = 1000
GRID = N // ROWS_BLK

NW = 32
NCH = 100
HCH = NCH // 2
CH = 100
NP = 10240
TILE_ROWS = NP // 16


def _gru(p0, p1, h, gh, wih, bih):
    agg = p0 + p1
    gi = jnp.dot(agg, wih, preferred_element_type=jnp.float32) + bih
    r = jax.nn.sigmoid(gi[:, 0:H] + gh[:, 0:H])
    z = jax.nn.sigmoid(gi[:, H:2 * H] + gh[:, H:2 * H])
    n = jnp.tanh(gi[:, 2 * H:3 * H] + r * gh[:, 2 * H:3 * H])
    return (1.0 - z) * n + z * h


def _tc_init(voro_x, W1t, Wc0, W_hht, b_hh2):
    def body(x_ref, w1_ref, wc_ref, whh_ref, bhh_ref, h_ref, m_ref, gh_ref):
        h = jnp.tanh(jnp.dot(x_ref[...], w1_ref[...],
                             preferred_element_type=jnp.float32))
        h_ref[...] = h
        m_ref[...] = jnp.dot(h, wc_ref[...], preferred_element_type=jnp.float32)
        gh_ref[...] = jnp.dot(h, whh_ref[...],
                              preferred_element_type=jnp.float32) + bhh_ref[...]

    return pl.pallas_call(
        body,
        grid=(GRID,),
        in_specs=[
            pl.BlockSpec((ROWS_BLK, F), lambda i: (i, 0)),
            pl.BlockSpec((F, H), lambda i: (0, 0)),
            pl.BlockSpec((H, H), lambda i: (0, 0)),
            pl.BlockSpec((H, 3 * H), lambda i: (0, 0)),
            pl.BlockSpec((1, 3 * H), lambda i: (0, 0)),
        ],
        out_specs=[
            pl.BlockSpec((ROWS_BLK, H), lambda i: (i, 0)),
            pl.BlockSpec((ROWS_BLK, H), lambda i: (i, 0)),
            pl.BlockSpec((ROWS_BLK, 3 * H), lambda i: (i, 0)),
        ],
        out_shape=[
            jax.ShapeDtypeStruct((N, H), jnp.float32),
            jax.ShapeDtypeStruct((N, H), jnp.float32),
            jax.ShapeDtypeStruct((N, 3 * H), jnp.float32),
        ],
    )(voro_x, W1t, Wc0, W_hht, b_hh2)


def _tc_step(p0, p1, h, gh, W_iht, b_ih2, Wc_next, W_hht, b_hh2):
    def body(p0_ref, p1_ref, h_ref, gh_ref, wih_ref, bih_ref, wc_ref,
             whh_ref, bhh_ref, hn_ref, m_ref, ghn_ref):
        h_new = _gru(p0_ref[...], p1_ref[...], h_ref[...], gh_ref[...],
                     wih_ref[...], bih_ref[...])
        hn_ref[...] = h_new
        m_ref[...] = jnp.dot(h_new, wc_ref[...],
                             preferred_element_type=jnp.float32)
        ghn_ref[...] = jnp.dot(h_new, whh_ref[...],
                               preferred_element_type=jnp.float32) + bhh_ref[...]

    return pl.pallas_call(
        body,
        grid=(GRID,),
        in_specs=[
            pl.BlockSpec((ROWS_BLK, H), lambda i: (i, 0)),
            pl.BlockSpec((ROWS_BLK, H), lambda i: (i, 0)),
            pl.BlockSpec((ROWS_BLK, H), lambda i: (i, 0)),
            pl.BlockSpec((ROWS_BLK, 3 * H), lambda i: (i, 0)),
            pl.BlockSpec((H, 3 * H), lambda i: (0, 0)),
            pl.BlockSpec((1, 3 * H), lambda i: (0, 0)),
            pl.BlockSpec((H, H), lambda i: (0, 0)),
            pl.BlockSpec((H, 3 * H), lambda i: (0, 0)),
            pl.BlockSpec((1, 3 * H), lambda i: (0, 0)),
        ],
        out_specs=[
            pl.BlockSpec((ROWS_BLK, H), lambda i: (i, 0)),
            pl.BlockSpec((ROWS_BLK, H), lambda i: (i, 0)),
            pl.BlockSpec((ROWS_BLK, 3 * H), lambda i: (i, 0)),
        ],
        out_shape=[
            jax.ShapeDtypeStruct((N, H), jnp.float32),
            jax.ShapeDtypeStruct((N, H), jnp.float32),
            jax.ShapeDtypeStruct((N, 3 * H), jnp.float32),
        ],
    )(p0, p1, h, gh, W_iht, b_ih2, Wc_next, W_hht, b_hh2)


def _tc_final(p0, p1, h, gh, W_iht, b_ih2, batch_r, Wp, bp2):
    def body(p0_ref, p1_ref, h_ref, gh_ref, wih_ref, bih_ref, b_ref,
             wp_ref, bp_ref, out_ref, sums_ref, cnt_ref):
        i = pl.program_id(0)
        h_new = _gru(p0_ref[...], p1_ref[...], h_ref[...], gh_ref[...],
                     wih_ref[...], bih_ref[...])
        seg = b_ref[0]
        oh = (lax.broadcasted_iota(jnp.int32, (G, ROWS_BLK), 0)
              == seg).astype(jnp.float32)

        @pl.when(i == 0)
        def _():
            sums_ref[...] = jnp.zeros_like(sums_ref)
            cnt_ref[...] = jnp.zeros_like(cnt_ref)

        sums_ref[...] += jnp.dot(oh, h_new, preferred_element_type=jnp.float32)
        cnt_ref[...] += jnp.sum(oh, axis=1, keepdims=True)

        @pl.when(i == GRID - 1)
        def _():
            pooled = sums_ref[...] / jnp.maximum(cnt_ref[...], 1.0)
            act = jnp.maximum(pooled, 0.0)
            pred = lax.dot_general(
                wp_ref[...], act, (((1,), (1,)), ((), ())),
                preferred_element_type=jnp.float32) + bp_ref[...]
            out_ref[...] = pred

    return pl.pallas_call(
        body,
        grid=(GRID,),
        in_specs=[
            pl.BlockSpec((ROWS_BLK, H), lambda i: (i, 0)),
            pl.BlockSpec((ROWS_BLK, H), lambda i: (i, 0)),
            pl.BlockSpec((ROWS_BLK, H), lambda i: (i, 0)),
            pl.BlockSpec((ROWS_BLK, 3 * H), lambda i: (i, 0)),
            pl.BlockSpec((H, 3 * H), lambda i: (0, 0)),
            pl.BlockSpec((1, 3 * H), lambda i: (0, 0)),
            pl.BlockSpec((1, 1, ROWS_BLK), lambda i: (i, 0, 0)),
            pl.BlockSpec((1, H), lambda i: (0, 0)),
            pl.BlockSpec((1, 1), lambda i: (0, 0)),
        ],
        out_specs=pl.BlockSpec((1, G), lambda i: (0, 0)),
        out_shape=jax.ShapeDtypeStruct((1, G), jnp.float32),
        scratch_shapes=[
            pltpu.VMEM((G, H), jnp.float32),
            pltpu.VMEM((G, 1), jnp.float32),
        ],
    )(p0, p1, h, gh, W_iht, b_ih2, batch_r, Wp, bp2)


def _sc_agg(m, srcr, dstr, zeros):
    mesh = plsc.VectorSubcoreMesh(core_axis_name="c", subcore_axis_name="s")

    @functools.partial(
        pl.kernel,
        out_type=[
            jax.ShapeDtypeStruct((NP, H), jnp.float32),
            jax.ShapeDtypeStruct((NP, H), jnp.float32),
        ],
        mesh=mesh,
        scratch_types=[
            pltpu.VMEM((HCH, CH), jnp.int32),
            pltpu.VMEM((HCH, CH), jnp.int32),
            pltpu.VMEM((2, CH, H), jnp.float32),
            pltpu.VMEM_SHARED((NP, H), jnp.float32),
            pltpu.SemaphoreType.DMA,
        ],
    )
    def agg_k(m_hbm, src_hbm, dst_hbm, z_hbm, out0_hbm, out1_hbm,
              src_v, dst_v, rows_v, acc_sh, sem):
        cid = lax.axis_index("c")
        sid = lax.axis_index("s")
        wid = sid * 2 + cid
        pltpu.sync_copy(z_hbm, acc_sh.at[pl.ds(sid * TILE_ROWS, TILE_ROWS)])
        plsc.subcore_barrier()

        buf_a = rows_v.at[0]
        buf_b = rows_v.at[1]

        def drain_one():
            pltpu.make_async_copy(m_hbm.at[src_v.at[0]], buf_a, sem).wait()

        def half_pass(half):
            pltpu.sync_copy(src_hbm.at[wid * 2 + half], src_v)
            pltpu.sync_copy(dst_hbm.at[wid * 2 + half], dst_v)
            pltpu.async_copy(m_hbm.at[src_v.at[0]], buf_a, sem)

            def body(k, c):
                j0 = 2 * k
                drain_one()
                pltpu.async_copy(m_hbm.at[src_v.at[j0 + 1]], buf_b, sem)
                pltpu.sync_copy(buf_a, acc_sh.at[dst_v.at[j0]], add=True)
                drain_one()

                @pl.when(k < HCH // 2 - 1)
                def _():
                    pltpu.async_copy(m_hbm.at[src_v.at[j0 + 2]], buf_a, sem)

                pltpu.sync_copy(buf_b, acc_sh.at[dst_v.at[j0 + 1]], add=True)
                return c

            lax.fori_loop(0, HCH // 2, body, 0)

        half_pass(0)
        half_pass(1)
        plsc.subcore_barrier()

        @pl.when(cid == 0)
        def _():
            pltpu.sync_copy(acc_sh.at[pl.ds(sid * TILE_ROWS, TILE_ROWS)],
                            out0_hbm.at[pl.ds(sid * TILE_ROWS, TILE_ROWS)])

        @pl.when(cid == 1)
        def _():
            pltpu.sync_copy(acc_sh.at[pl.ds(sid * TILE_ROWS, TILE_ROWS)],
                            out1_hbm.at[pl.ds(sid * TILE_ROWS, TILE_ROWS)])

    return agg_k(m, srcr, dstr, zeros)


def kernel(voro_x, edge_index, batch, W1, Wconv, W_ih, W_hh, b_ih, b_hh, Wp, bp):
    W1t = W1.T
    W_iht = W_ih.T
    W_hht = W_hh.T
    b_ih2 = b_ih.reshape(1, 3 * H)
    b_hh2 = b_hh.reshape(1, 3 * H)
    src = edge_index[0].astype(jnp.int32).reshape(NW * 2, HCH, CH)
    dst = edge_index[1].astype(jnp.int32).reshape(NW * 2, HCH, CH)
    zeros = jnp.zeros((TILE_ROWS, H), jnp.float32)
    batch_r = batch.astype(jnp.int32).reshape(GRID, 1, ROWS_BLK)

    h, m, gh = _tc_init(voro_x, W1t, Wconv[0], W_hht, b_hh2)
    out2 = None
    for i in range(STEPS):
        p0, p1 = _sc_agg(m, src, dst, zeros)
        if i < STEPS - 1:
            h, m, gh = _tc_step(p0, p1, h, gh, W_iht, b_ih2, Wconv[i + 1],
                                W_hht, b_hh2)
        else:
            out2 = _tc_final(p0, p1, h, gh, W_iht, b_ih2, batch_r, Wp,
                             bp.reshape(1, 1))
    return out2.reshape(G)


# speedup vs baseline: 8.8913x; 1.0509x over previous
import functools

import jax
import jax.numpy as jnp
from jax import lax
from jax.experimental import pallas as pl
from jax.experimental.pallas import tpu as pltpu
from jax.experimental.pallas import tpu_sc as plsc

N = 10000
E = 320000
F = 128
H = 128
STEPS = 4
G = 64

ROWS_BLK = 1000
GRID = N // ROWS_BLK

NW = 32
NCH = 100
HCH = NCH // 2
CH = 100
NP = 10240
TILE_ROWS = NP // 16


def _gru(p0, p1, h, wih, bih, whh, bhh):
    agg = p0 + p1
    gi = jnp.dot(agg, wih, preferred_element_type=jnp.float32) + bih
    gh = jnp.dot(h, whh, preferred_element_type=jnp.float32) + bhh
    r = jax.nn.sigmoid(gi[:, 0:H] + gh[:, 0:H])
    z = jax.nn.sigmoid(gi[:, H:2 * H] + gh[:, H:2 * H])
    n = jnp.tanh(gi[:, 2 * H:3 * H] + r * gh[:, 2 * H:3 * H])
    return (1.0 - z) * n + z * h


def _tc_init(voro_x, W1t, Wc0):
    def body(x_ref, w1_ref, wc_ref, h_ref, m_ref):
        h = jnp.tanh(jnp.dot(x_ref[...], w1_ref[...],
                             preferred_element_type=jnp.float32))
        h_ref[...] = h
        m_ref[...] = jnp.dot(h, wc_ref[...], preferred_element_type=jnp.float32)

    return pl.pallas_call(
        body,
        grid=(GRID,),
        in_specs=[
            pl.BlockSpec((ROWS_BLK, F), lambda i: (i, 0)),
            pl.BlockSpec((F, H), lambda i: (0, 0)),
            pl.BlockSpec((H, H), lambda i: (0, 0)),
        ],
        out_specs=[
            pl.BlockSpec((ROWS_BLK, H), lambda i: (i, 0)),
            pl.BlockSpec((ROWS_BLK, H), lambda i: (i, 0)),
        ],
        out_shape=[
            jax.ShapeDtypeStruct((N, H), jnp.float32),
            jax.ShapeDtypeStruct((N, H), jnp.float32),
        ],
    )(voro_x, W1t, Wc0)


def _tc_step(p0, p1, h, W_iht, b_ih2, Wc_next, W_hht, b_hh2):
    def body(p0_ref, p1_ref, h_ref, wih_ref, bih_ref, wc_ref,
             whh_ref, bhh_ref, hn_ref, m_ref):
        h_new = _gru(p0_ref[...], p1_ref[...], h_ref[...],
                     wih_ref[...], bih_ref[...], whh_ref[...], bhh_ref[...])
        hn_ref[...] = h_new
        m_ref[...] = jnp.dot(h_new, wc_ref[...],
                             preferred_element_type=jnp.float32)

    return pl.pallas_call(
        body,
        grid=(GRID,),
        in_specs=[
            pl.BlockSpec((ROWS_BLK, H), lambda i: (i, 0)),
            pl.BlockSpec((ROWS_BLK, H), lambda i: (i, 0)),
            pl.BlockSpec((ROWS_BLK, H), lambda i: (i, 0)),
            pl.BlockSpec((H, 3 * H), lambda i: (0, 0)),
            pl.BlockSpec((1, 3 * H), lambda i: (0, 0)),
            pl.BlockSpec((H, H), lambda i: (0, 0)),
            pl.BlockSpec((H, 3 * H), lambda i: (0, 0)),
            pl.BlockSpec((1, 3 * H), lambda i: (0, 0)),
        ],
        out_specs=[
            pl.BlockSpec((ROWS_BLK, H), lambda i: (i, 0)),
            pl.BlockSpec((ROWS_BLK, H), lambda i: (i, 0)),
        ],
        out_shape=[
            jax.ShapeDtypeStruct((N, H), jnp.float32),
            jax.ShapeDtypeStruct((N, H), jnp.float32),
        ],
    )(p0, p1, h, W_iht, b_ih2, Wc_next, W_hht, b_hh2)


def _tc_final(p0, p1, h, W_iht, b_ih2, W_hht, b_hh2, batch_r, Wp, bp2):
    def body(p0_ref, p1_ref, h_ref, wih_ref, bih_ref, whh_ref, bhh_ref,
             b_ref, wp_ref, bp_ref, out_ref, sums_ref, cnt_ref):
        i = pl.program_id(0)
        h_new = _gru(p0_ref[...], p1_ref[...], h_ref[...],
                     wih_ref[...], bih_ref[...], whh_ref[...], bhh_ref[...])
        seg = b_ref[0]
        oh = (lax.broadcasted_iota(jnp.int32, (G, ROWS_BLK), 0)
              == seg).astype(jnp.float32)

        @pl.when(i == 0)
        def _():
            sums_ref[...] = jnp.zeros_like(sums_ref)
            cnt_ref[...] = jnp.zeros_like(cnt_ref)

        sums_ref[...] += jnp.dot(oh, h_new, preferred_element_type=jnp.float32)
        cnt_ref[...] += jnp.sum(oh, axis=1, keepdims=True)

        @pl.when(i == GRID - 1)
        def _():
            pooled = sums_ref[...] / jnp.maximum(cnt_ref[...], 1.0)
            act = jnp.maximum(pooled, 0.0)
            pred = lax.dot_general(
                wp_ref[...], act, (((1,), (1,)), ((), ())),
                preferred_element_type=jnp.float32) + bp_ref[...]
            out_ref[...] = pred

    return pl.pallas_call(
        body,
        grid=(GRID,),
        in_specs=[
            pl.BlockSpec((ROWS_BLK, H), lambda i: (i, 0)),
            pl.BlockSpec((ROWS_BLK, H), lambda i: (i, 0)),
            pl.BlockSpec((ROWS_BLK, H), lambda i: (i, 0)),
            pl.BlockSpec((H, 3 * H), lambda i: (0, 0)),
            pl.BlockSpec((1, 3 * H), lambda i: (0, 0)),
            pl.BlockSpec((H, 3 * H), lambda i: (0, 0)),
            pl.BlockSpec((1, 3 * H), lambda i: (0, 0)),
            pl.BlockSpec((1, 1, ROWS_BLK), lambda i: (i, 0, 0)),
            pl.BlockSpec((1, H), lambda i: (0, 0)),
            pl.BlockSpec((1, 1), lambda i: (0, 0)),
        ],
        out_specs=pl.BlockSpec((1, G), lambda i: (0, 0)),
        out_shape=jax.ShapeDtypeStruct((1, G), jnp.float32),
        scratch_shapes=[
            pltpu.VMEM((G, H), jnp.float32),
            pltpu.VMEM((G, 1), jnp.float32),
        ],
    )(p0, p1, h, W_iht, b_ih2, W_hht, b_hh2, batch_r, Wp, bp2)


def _sc_agg(m, srcr, dstr, zeros):
    mesh = plsc.VectorSubcoreMesh(core_axis_name="c", subcore_axis_name="s")

    @functools.partial(
        pl.kernel,
        out_type=[
            jax.ShapeDtypeStruct((NP, H), jnp.float32),
            jax.ShapeDtypeStruct((NP, H), jnp.float32),
        ],
        mesh=mesh,
        scratch_types=[
            pltpu.VMEM((HCH, CH), jnp.int32),
            pltpu.VMEM((HCH, CH), jnp.int32),
            pltpu.VMEM((2, CH, H), jnp.float32),
            pltpu.VMEM_SHARED((NP, H), jnp.float32),
            pltpu.SemaphoreType.DMA,
        ],
    )
    def agg_k(m_hbm, src_hbm, dst_hbm, z_hbm, out0_hbm, out1_hbm,
              src_v, dst_v, rows_v, acc_sh, sem):
        cid = lax.axis_index("c")
        sid = lax.axis_index("s")
        wid = sid * 2 + cid
        pltpu.sync_copy(z_hbm, acc_sh.at[pl.ds(sid * TILE_ROWS, TILE_ROWS)])
        plsc.subcore_barrier()

        buf_a = rows_v.at[0]
        buf_b = rows_v.at[1]

        def drain_one():
            pltpu.make_async_copy(m_hbm.at[src_v.at[0]], buf_a, sem).wait()

        def half_pass(half):
            pltpu.sync_copy(src_hbm.at[wid * 2 + half], src_v)
            pltpu.sync_copy(dst_hbm.at[wid * 2 + half], dst_v)
            pltpu.async_copy(m_hbm.at[src_v.at[0]], buf_a, sem)

            def body(k, c):
                j0 = 2 * k
                drain_one()
                pltpu.async_copy(m_hbm.at[src_v.at[j0 + 1]], buf_b, sem)
                pltpu.sync_copy(buf_a, acc_sh.at[dst_v.at[j0]], add=True)
                drain_one()

                @pl.when(k < HCH // 2 - 1)
                def _():
                    pltpu.async_copy(m_hbm.at[src_v.at[j0 + 2]], buf_a, sem)

                pltpu.sync_copy(buf_b, acc_sh.at[dst_v.at[j0 + 1]], add=True)
                return c

            lax.fori_loop(0, HCH // 2, body, 0)

        half_pass(0)
        half_pass(1)
        plsc.subcore_barrier()

        @pl.when(cid == 0)
        def _():
            pltpu.sync_copy(acc_sh.at[pl.ds(sid * TILE_ROWS, TILE_ROWS)],
                            out0_hbm.at[pl.ds(sid * TILE_ROWS, TILE_ROWS)])

        @pl.when(cid == 1)
        def _():
            pltpu.sync_copy(acc_sh.at[pl.ds(sid * TILE_ROWS, TILE_ROWS)],
                            out1_hbm.at[pl.ds(sid * TILE_ROWS, TILE_ROWS)])

    return agg_k(m, srcr, dstr, zeros)


def kernel(voro_x, edge_index, batch, W1, Wconv, W_ih, W_hh, b_ih, b_hh, Wp, bp):
    W1t = W1.T
    W_iht = W_ih.T
    W_hht = W_hh.T
    b_ih2 = b_ih.reshape(1, 3 * H)
    b_hh2 = b_hh.reshape(1, 3 * H)
    src = edge_index[0].astype(jnp.int32).reshape(NW * 2, HCH, CH)
    dst = edge_index[1].astype(jnp.int32).reshape(NW * 2, HCH, CH)
    zeros = jnp.zeros((TILE_ROWS, H), jnp.float32)
    batch_r = batch.astype(jnp.int32).reshape(GRID, 1, ROWS_BLK)

    h, m = _tc_init(voro_x, W1t, Wconv[0])
    out2 = None
    for i in range(STEPS):
        p0, p1 = _sc_agg(m, src, dst, zeros)
        if i < STEPS - 1:
            h, m = _tc_step(p0, p1, h, W_iht, b_ih2, Wconv[i + 1],
                            W_hht, b_hh2)
        else:
            out2 = _tc_final(p0, p1, h, W_iht, b_ih2, W_hht, b_hh2,
                             batch_r, Wp, bp.reshape(1, 1))
    return out2.reshape(G)
